# baseline (speedup 1.0000x reference)
import functools

import jax
import jax.numpy as jnp
from jax import lax
from jax.experimental import pallas as pl
from jax.experimental.pallas import tpu as pltpu
from jax.experimental.pallas import tpu_sc as plsc

N_NODES = 10000
N_EDGES = 320000
D_NODE = 142
D_HID = 128
NUM_GRAPHS = 25
NUM_ENC = 10

NC = 2
NS = 16
NW = NC * NS
EPW = N_EDGES // NW
K = 40
NB = EPW // K
RSTRIPE = 640
RCHUNK = 40


def _p_body(atom_ref, w_ref, o_ref):
    o_ref[...] = jnp.dot(atom_ref[...], w_ref[...],
                         preferred_element_type=jnp.float32)


def _compute_p(atom_feature, w_top):
    return pl.pallas_call(
        _p_body,
        out_shape=jax.ShapeDtypeStruct((N_NODES, D_HID), jnp.float32),
    )(atom_feature, w_top)


_DB = 400
_EB = 4 * _DB
_NQB = N_EDGES // _EB

_INV2PI = float(1.0 / (2.0 * 3.14159265358979323846))
_TWOPI = float(2.0 * 3.14159265358979323846)


def _q_body(din_ref, sc32_ref, bond_ref, wb_ref, m_ref, bias_ref, o_ref):
    x = din_ref[...] * sc32_ref[...]
    y = x - jnp.round(x * _INV2PI) * _TWOPI
    h = y * 0.5
    h2 = h * h
    sh = h * (1.0 + h2 * (-1.0 / 6.0 + h2 * (1.0 / 120.0 + h2 * (
        -1.0 / 5040.0 + h2 * (1.0 / 362880.0)))))
    ch = 1.0 + h2 * (-0.5 + h2 * (1.0 / 24.0 + h2 * (-1.0 / 720.0 + h2 * (
        1.0 / 40320.0 + h2 * (-1.0 / 3628800.0)))))
    sy = 2.0 * sh * ch
    cy = 1.0 - 2.0 * sh * sh
    lane = lax.broadcasted_iota(jnp.int32, (_DB, 128), 1) % 32
    trig = jnp.where(lane < 10, sy, cy)
    base = jnp.dot(bond_ref[...], wb_ref[...],
                   preferred_element_type=jnp.float32) + bias_ref[...]
    for g in range(4):
        qg = jnp.dot(trig, m_ref[g], preferred_element_type=jnp.float32)
        o_ref[pl.ds(g * _DB, _DB), :] = qg + base[g * _DB:(g + 1) * _DB]


def _compute_q(dist32, scale32, bond_p, w_bond, m_blocks, bias_row):
    return pl.pallas_call(
        _q_body,
        grid=(_NQB,),
        in_specs=[
            pl.BlockSpec((_DB, 128), lambda i: (i, 0)),
            pl.BlockSpec((1, 128), lambda i: (0, 0)),
            pl.BlockSpec((_EB, 5), lambda i: (i, 0)),
            pl.BlockSpec((5, D_HID), lambda i: (0, 0)),
            pl.BlockSpec((4, 128, D_HID), lambda i: (0, 0, 0)),
            pl.BlockSpec((1, D_HID), lambda i: (0, 0)),
        ],
        out_specs=pl.BlockSpec((_EB, D_HID), lambda i: (i, 0)),
        out_shape=jax.ShapeDtypeStruct((N_EDGES, D_HID), jnp.float32),
    )(dist32, scale32, bond_p, w_bond, m_blocks, bias_row)


DEPTH = 3


def _sc_agg_body(p_hbm, q_hbm, src_hbm, dst_hbm, out_hbm,
                 src_v, dring, qb, rows, sem_q, sem_g, sem_d, sem_sc, agg_sh):
    c = lax.axis_index("c")
    s = lax.axis_index("s")
    wid = c * NS + s
    base_r = s * RSTRIPE
    n_chunks = jnp.where(s == NS - 1, 10, 16)
    ebase = wid * EPW

    pltpu.sync_copy(src_hbm.at[pl.ds(ebase, EPW)], src_v)

    def _zero(j, _):
        rows[0][j // 8, pl.ds((j % 8) * 16, 16)] = jnp.zeros((16,), jnp.float32)
        return 0
    lax.fori_loop(0, RCHUNK * 8, _zero, 0)

    def _zinit(i, _):
        pltpu.sync_copy(rows[0], agg_sh.at[pl.ds(base_r + i * RCHUNK, RCHUNK)])
        return 0
    lax.fori_loop(0, n_chunks, _zinit, 0)
    plsc.subcore_barrier()

    def _q_desc(b, slot):
        return pltpu.make_async_copy(
            q_hbm.at[pl.ds(ebase + b * K, K)], qb[slot], sem_q[slot])

    def _g_desc(b, slot):
        return pltpu.make_async_copy(
            p_hbm.at[src_v.at[pl.ds(b * K, K)]], rows[slot], sem_g[slot])

    def _d_desc(b, slot):
        return pltpu.make_async_copy(
            dst_hbm.at[pl.ds(ebase + b * K, K)], dring[slot], sem_d[slot])

    def _sc_desc(b, slot):
        return pltpu.make_async_copy(rows[slot], agg_sh.at[dring[slot]],
                                     sem_sc[slot])

    def _compute(b, j):
        def _relu_row(e, _):
            for kk in range(D_HID // 16):
                sl = pl.ds(kk * 16, 16)
                rows[j][e, sl] = jnp.maximum(rows[j][e, sl] + qb[j][e, sl], 0.0)
            return 0
        lax.fori_loop(0, K, _relu_row, 0)

    _q_desc(0, 0).start()
    _g_desc(0, 0).start()
    _d_desc(0, 0).start()

    def _outer(o, _):
        for j in range(DEPTH):
            b = o * DEPTH + j

            @pl.when(b >= 2)
            def _():
                _sc_desc(b - 2, (j + 1) % DEPTH).wait()

            _q_desc(b + 1, (j + 1) % DEPTH).start()
            _g_desc(b + 1, (j + 1) % DEPTH).start()
            _d_desc(b + 1, (j + 1) % DEPTH).start()

            _q_desc(b, j).wait()
            _g_desc(b, j).wait()
            _compute(b, j)
            _d_desc(b, j).wait()
            _sc_desc(b, j).start(add=True)
        return 0
    lax.fori_loop(0, (NB - 1) // DEPTH, _outer, 0)

    bl = NB - 1
    jl = bl % DEPTH
    _sc_desc(bl - 2, (jl + 1) % DEPTH).wait()
    _q_desc(bl, jl).wait()
    _g_desc(bl, jl).wait()
    _compute(bl, jl)
    _d_desc(bl, jl).wait()
    _sc_desc(bl, jl).start(add=True)
    _sc_desc(bl - 1, (bl - 1) % DEPTH).wait()
    _sc_desc(bl, jl).wait()
    plsc.subcore_barrier()

    def _out(i, _):
        r0 = base_r + i * RCHUNK
        pltpu.sync_copy(agg_sh.at[pl.ds(r0, RCHUNK)], rows[0])
        pltpu.sync_copy(rows[0], out_hbm.at[c, pl.ds(r0, RCHUNK)])
        return 0
    lax.fori_loop(0, n_chunks, _out, 0)


@functools.cache
def _get_sc_agg():
  return functools.partial(
    pl.kernel,
    out_type=jax.ShapeDtypeStruct((NC, N_NODES, D_HID), jnp.float32),
    mesh=plsc.VectorSubcoreMesh(core_axis_name="c", subcore_axis_name="s",
                                num_cores=NC, num_subcores=NS),
    scratch_types=[
        pltpu.VMEM((EPW,), jnp.int32),
        [pltpu.VMEM((K,), jnp.int32)] * DEPTH,
        [pltpu.VMEM((K, D_HID), jnp.float32)] * DEPTH,
        [pltpu.VMEM((K, D_HID), jnp.float32)] * DEPTH,
        [pltpu.SemaphoreType.DMA] * DEPTH,
        [pltpu.SemaphoreType.DMA] * DEPTH,
        [pltpu.SemaphoreType.DMA] * DEPTH,
        [pltpu.SemaphoreType.DMA] * DEPTH,
        pltpu.VMEM_SHARED((N_NODES, D_HID), jnp.float32),
    ],
  )(_sc_agg_body)


def _final_body(atom_ref, agg_ref, n2g_ref, wnt_ref, wnb_ref, bn_ref,
                wo_ref, bo_ref, o_ref):
    agg = agg_ref[0] + agg_ref[1]
    h = jnp.dot(atom_ref[...], wnt_ref[...], preferred_element_type=jnp.float32)
    h = h + jnp.dot(agg, wnb_ref[...], preferred_element_type=jnp.float32)
    h = jnp.maximum(h + bn_ref[...], 0.0)
    gids = lax.broadcasted_iota(jnp.int32, (N_NODES, D_HID), 1)
    mask = (n2g_ref[...] == gids).astype(jnp.float32)
    dn = (((0,), (0,)), ((), ()))
    gsum = lax.dot_general(mask, h, dn, preferred_element_type=jnp.float32)
    ones = jnp.ones((N_NODES, 1), jnp.float32)
    gcnt = lax.dot_general(mask, ones, dn, preferred_element_type=jnp.float32)
    gmean = gsum / jnp.maximum(gcnt, 1.0)
    h = h + jnp.dot(mask, gmean, preferred_element_type=jnp.float32)
    logits = jnp.dot(h, wo_ref[...], preferred_element_type=jnp.float32) + bo_ref[...]
    o_ref[...] = 1.0 / (1.0 + jnp.exp(-logits))


def _compute_out(atom_feature, agg2, n2g_col, wn_top, wn_bot, bn_row, w_out, bo_row):
    return pl.pallas_call(
        _final_body,
        out_shape=jax.ShapeDtypeStruct((N_NODES, 1), jnp.float32),
    )(atom_feature, agg2, n2g_col, wn_top, wn_bot, bn_row, w_out, bo_row)


def kernel(atom_feature, edge_index, bond_feature, distance, b_factor, node2graph,
           W_msg, b_msg, W_node, b_node, W_out, b_out):
    w_atom = W_msg[:D_NODE]
    w_bond = W_msg[D_NODE:D_NODE + 5]
    w_sin = W_msg[D_NODE + 5:D_NODE + 5 + NUM_ENC]
    w_cos = W_msg[D_NODE + 5 + NUM_ENC:]
    def _perm(x):
        new_shape = (_NQB, _DB, 4) + x.shape[1:]
        axes = (0, 2, 1) + tuple(range(3, x.ndim + 2))
        return x.reshape(new_shape).transpose(axes).reshape(x.shape)

    dist32 = jnp.repeat(distance, 32).reshape(-1, 128)
    k32 = jnp.arange(128) % 32
    scale32 = jnp.where(k32 < 10, 2.0 ** (-k32.astype(jnp.float32)),
                        jnp.where(k32 < 20,
                                  2.0 ** (-(k32 - 10).astype(jnp.float32)),
                                  0.0)).reshape(1, 128)
    wsc = jnp.concatenate([w_sin, w_cos], axis=0)
    m_blocks = jnp.stack([jnp.pad(wsc, ((32 * g, 108 - 32 * g), (0, 0)))
                          for g in range(4)])

    p = _compute_p(atom_feature, w_atom)
    q = _compute_q(dist32, scale32, _perm(bond_feature),
                   w_bond, m_blocks, b_msg[None, :])
    agg2 = _get_sc_agg()(p, q, _perm(edge_index[0]), _perm(edge_index[1]))
    out2 = _compute_out(atom_feature, agg2, node2graph[:, None],
                        W_node[:D_NODE], W_node[D_NODE:], b_node[None, :],
                        W_out, b_out[None, :])
    return (out2[:, 0], b_factor)


# speedup vs baseline: 3.0711x; 3.0711x over previous
import functools

import jax
import jax.numpy as jnp
from jax import lax
from jax.experimental import pallas as pl
from jax.experimental.pallas import tpu as pltpu
from jax.experimental.pallas import tpu_sc as plsc

N_NODES = 10000
N_EDGES = 320000
D_NODE = 142
D_HID = 128
NUM_GRAPHS = 25
NUM_ENC = 10

NC = 2
NS = 16
NW = NC * NS
EPW = N_EDGES // NW
K = 40
NB = EPW // K
RSTRIPE = 640
RCHUNK = 40


def _p_body(atom_ref, w_ref, o_ref):
    o_ref[...] = jnp.dot(atom_ref[...], w_ref[...],
                         preferred_element_type=jnp.float32)


def _compute_p(atom_feature, w_top):
    return pl.pallas_call(
        _p_body,
        out_shape=jax.ShapeDtypeStruct((N_NODES, D_HID), jnp.float32),
    )(atom_feature, w_top)


_EB = 1600
_NQB = N_EDGES // _EB

_INV2PI = float(1.0 / (2.0 * 3.14159265358979323846))
_TWOPI = float(2.0 * 3.14159265358979323846)


def _q_body(dist_ref, sc32_ref, bond_ref, wb_ref, w32_ref, bias_ref, o_ref):
    x = dist_ref[...] * sc32_ref[...]
    y = x - jnp.round(x * _INV2PI) * _TWOPI
    h = y * 0.5
    h2 = h * h
    sh = h * (1.0 + h2 * (-1.0 / 6.0 + h2 * (1.0 / 120.0 + h2 * (
        -1.0 / 5040.0 + h2 * (1.0 / 362880.0)))))
    ch = 1.0 + h2 * (-0.5 + h2 * (1.0 / 24.0 + h2 * (-1.0 / 720.0 + h2 * (
        1.0 / 40320.0 + h2 * (-1.0 / 3628800.0)))))
    sy = 2.0 * sh * ch
    cy = 1.0 - 2.0 * sh * sh
    lane = lax.broadcasted_iota(jnp.int32, (_EB, 32), 1)
    trig = jnp.where(lane < 10, sy, cy)
    acc = jnp.dot(trig, w32_ref[...], preferred_element_type=jnp.float32)
    acc = acc + jnp.dot(bond_ref[...], wb_ref[...],
                        preferred_element_type=jnp.float32)
    o_ref[...] = acc + bias_ref[...]


def _compute_q(dist_col, scale32, bond, w_bond, w32, bias_row):
    return pl.pallas_call(
        _q_body,
        grid=(_NQB,),
        in_specs=[
            pl.BlockSpec((_EB, 1), lambda i: (i, 0)),
            pl.BlockSpec((1, 32), lambda i: (0, 0)),
            pl.BlockSpec((_EB, 5), lambda i: (i, 0)),
            pl.BlockSpec((5, D_HID), lambda i: (0, 0)),
            pl.BlockSpec((32, D_HID), lambda i: (0, 0)),
            pl.BlockSpec((1, D_HID), lambda i: (0, 0)),
        ],
        out_specs=pl.BlockSpec((_EB, D_HID), lambda i: (i, 0)),
        out_shape=jax.ShapeDtypeStruct((N_EDGES, D_HID), jnp.float32),
    )(dist_col, scale32, bond, w_bond, w32, bias_row)


DEPTH = 3


def _sc_agg_body(p_hbm, q_hbm, src_hbm, dst_hbm, out_hbm,
                 src_v, dring, qb, rows, sem_q, sem_g, sem_d, sem_sc, agg_sh):
    c = lax.axis_index("c")
    s = lax.axis_index("s")
    wid = c * NS + s
    base_r = s * RSTRIPE
    n_chunks = jnp.where(s == NS - 1, 10, 16)
    ebase = wid * EPW

    pltpu.sync_copy(src_hbm.at[pl.ds(ebase, EPW)], src_v)

    def _zero(j, _):
        rows[0][j // 8, pl.ds((j % 8) * 16, 16)] = jnp.zeros((16,), jnp.float32)
        return 0
    lax.fori_loop(0, RCHUNK * 8, _zero, 0)

    def _zinit(i, _):
        pltpu.sync_copy(rows[0], agg_sh.at[pl.ds(base_r + i * RCHUNK, RCHUNK)])
        return 0
    lax.fori_loop(0, n_chunks, _zinit, 0)
    plsc.subcore_barrier()

    def _q_desc(b, slot):
        return pltpu.make_async_copy(
            q_hbm.at[pl.ds(ebase + b * K, K)], qb[slot], sem_q[slot])

    def _g_desc(b, slot):
        return pltpu.make_async_copy(
            p_hbm.at[src_v.at[pl.ds(b * K, K)]], rows[slot], sem_g[slot])

    def _d_desc(b, slot):
        return pltpu.make_async_copy(
            dst_hbm.at[pl.ds(ebase + b * K, K)], dring[slot], sem_d[slot])

    def _sc_desc(b, slot):
        return pltpu.make_async_copy(rows[slot], agg_sh.at[dring[slot]],
                                     sem_sc[slot])

    def _compute(b, j):
        def _relu_row(e, _):
            for kk in range(D_HID // 16):
                sl = pl.ds(kk * 16, 16)
                rows[j][e, sl] = jnp.maximum(rows[j][e, sl] + qb[j][e, sl], 0.0)
            return 0
        lax.fori_loop(0, K, _relu_row, 0)

    _q_desc(0, 0).start()
    _g_desc(0, 0).start()
    _d_desc(0, 0).start()

    def _outer(o, _):
        for j in range(DEPTH):
            b = o * DEPTH + j

            @pl.when(b >= 2)
            def _():
                _sc_desc(b - 2, (j + 1) % DEPTH).wait()

            _q_desc(b + 1, (j + 1) % DEPTH).start()
            _g_desc(b + 1, (j + 1) % DEPTH).start()
            _d_desc(b + 1, (j + 1) % DEPTH).start()

            _q_desc(b, j).wait()
            _g_desc(b, j).wait()
            _compute(b, j)
            _d_desc(b, j).wait()
            _sc_desc(b, j).start(add=True)
        return 0
    lax.fori_loop(0, (NB - 1) // DEPTH, _outer, 0)

    bl = NB - 1
    jl = bl % DEPTH
    _sc_desc(bl - 2, (jl + 1) % DEPTH).wait()
    _q_desc(bl, jl).wait()
    _g_desc(bl, jl).wait()
    _compute(bl, jl)
    _d_desc(bl, jl).wait()
    _sc_desc(bl, jl).start(add=True)
    _sc_desc(bl - 1, (bl - 1) % DEPTH).wait()
    _sc_desc(bl, jl).wait()
    plsc.subcore_barrier()

    def _out(i, _):
        r0 = base_r + i * RCHUNK
        pltpu.sync_copy(agg_sh.at[pl.ds(r0, RCHUNK)], rows[0])
        pltpu.sync_copy(rows[0], out_hbm.at[c, pl.ds(r0, RCHUNK)])
        return 0
    lax.fori_loop(0, n_chunks, _out, 0)


@functools.cache
def _get_sc_agg():
  return functools.partial(
    pl.kernel,
    out_type=jax.ShapeDtypeStruct((NC, N_NODES, D_HID), jnp.float32),
    mesh=plsc.VectorSubcoreMesh(core_axis_name="c", subcore_axis_name="s",
                                num_cores=NC, num_subcores=NS),
    scratch_types=[
        pltpu.VMEM((EPW,), jnp.int32),
        [pltpu.VMEM((K,), jnp.int32)] * DEPTH,
        [pltpu.VMEM((K, D_HID), jnp.float32)] * DEPTH,
        [pltpu.VMEM((K, D_HID), jnp.float32)] * DEPTH,
        [pltpu.SemaphoreType.DMA] * DEPTH,
        [pltpu.SemaphoreType.DMA] * DEPTH,
        [pltpu.SemaphoreType.DMA] * DEPTH,
        [pltpu.SemaphoreType.DMA] * DEPTH,
        pltpu.VMEM_SHARED((N_NODES, D_HID), jnp.float32),
    ],
  )(_sc_agg_body)


def _final_body(atom_ref, agg_ref, n2g_ref, wnt_ref, wnb_ref, bn_ref,
                wo_ref, bo_ref, o_ref):
    agg = agg_ref[0] + agg_ref[1]
    h = jnp.dot(atom_ref[...], wnt_ref[...], preferred_element_type=jnp.float32)
    h = h + jnp.dot(agg, wnb_ref[...], preferred_element_type=jnp.float32)
    h = jnp.maximum(h + bn_ref[...], 0.0)
    gids = lax.broadcasted_iota(jnp.int32, (N_NODES, D_HID), 1)
    mask = (n2g_ref[...] == gids).astype(jnp.float32)
    dn = (((0,), (0,)), ((), ()))
    gsum = lax.dot_general(mask, h, dn, preferred_element_type=jnp.float32)
    ones = jnp.ones((N_NODES, 1), jnp.float32)
    gcnt = lax.dot_general(mask, ones, dn, preferred_element_type=jnp.float32)
    gmean = gsum / jnp.maximum(gcnt, 1.0)
    h = h + jnp.dot(mask, gmean, preferred_element_type=jnp.float32)
    logits = jnp.dot(h, wo_ref[...], preferred_element_type=jnp.float32) + bo_ref[...]
    o_ref[...] = 1.0 / (1.0 + jnp.exp(-logits))


def _compute_out(atom_feature, agg2, n2g_col, wn_top, wn_bot, bn_row, w_out, bo_row):
    return pl.pallas_call(
        _final_body,
        out_shape=jax.ShapeDtypeStruct((N_NODES, 1), jnp.float32),
    )(atom_feature, agg2, n2g_col, wn_top, wn_bot, bn_row, w_out, bo_row)


def kernel(atom_feature, edge_index, bond_feature, distance, b_factor, node2graph,
           W_msg, b_msg, W_node, b_node, W_out, b_out):
    w_atom = W_msg[:D_NODE]
    w_bond = W_msg[D_NODE:D_NODE + 5]
    w_sin = W_msg[D_NODE + 5:D_NODE + 5 + NUM_ENC]
    w_cos = W_msg[D_NODE + 5 + NUM_ENC:]
    k32 = jnp.arange(32)
    scale32 = jnp.where(k32 < 10, 2.0 ** (-k32.astype(jnp.float32)),
                        jnp.where(k32 < 20,
                                  2.0 ** (-(k32 - 10).astype(jnp.float32)),
                                  0.0)).reshape(1, 32)
    w32 = jnp.pad(jnp.concatenate([w_sin, w_cos], axis=0),
                  ((0, 12), (0, 0)))

    p = _compute_p(atom_feature, w_atom)
    q = _compute_q(distance[:, None], scale32, bond_feature,
                   w_bond, w32, b_msg[None, :])
    agg2 = _get_sc_agg()(p, q, edge_index[0], edge_index[1])
    out2 = _compute_out(atom_feature, agg2, node2graph[:, None],
                        W_node[:D_NODE], W_node[D_NODE:], b_node[None, :],
                        W_out, b_out[None, :])
    return (out2[:, 0], b_factor)
